# Initial kernel scaffold; baseline (speedup 1.0000x reference)
import jax
import jax.numpy as jnp
from jax.experimental import pallas as pl


def kernel(x, W_proj, node_weights, rms_weight, base_thresh, adapt_strength):
    raise NotImplementedError("write your pallas kernel here")


# speedup vs baseline: 1.1635x; 1.1635x over previous
import math

import jax
import jax.numpy as jnp
from jax.experimental import pallas as pl
from jax.experimental.pallas import tpu as pltpu

TREE_DEPTH = 4
D_MODEL = 1024
NUM_NODES = (1 << TREE_DEPTH) - 1
TAU_MEM = 0.99
TAU_ADAPT = 0.95
RMS_EPS = 1.1920929e-07


def _tree_mix_kernel(x_ref, w_ref, nw_ref, bt_ref, as_ref, mix_ref, leaves_ref):
    i = pl.program_id(0)
    nblk = pl.num_programs(0)
    blk = x_ref[...]
    leaves_ref[:, i, :] = jnp.mean(blk, axis=1)

    @pl.when(i == nblk - 1)
    def _tree():
        B = blk.shape[0]
        D = blk.shape[2]
        leaves = leaves_ref[...]
        states = [None] * NUM_NODES
        for n in range(7, 15):
            states[n] = leaves[:, n - 7, :]
        w = w_ref[...]
        for level in range(2, -1, -1):
            lo = (1 << level) - 1
            hi = (1 << (level + 1)) - 1
            nodes = list(range(lo, hi))
            nlev = len(nodes)
            l_st = jnp.stack([states[2 * n + 1] for n in nodes], axis=1)
            r_st = jnp.stack([states[2 * n + 2] for n in nodes], axis=1)
            fused = jnp.concatenate([l_st, r_st], axis=-1)
            fused2 = fused.reshape(B * nlev, 2 * D)
            proj = jax.lax.dot_general(
                fused2, w, (((1,), (1,)), ((), ())),
                preferred_element_type=jnp.float32,
            ).reshape(B, nlev, D)
            bt = bt_ref[level, :]
            asw = as_ref[level, :]
            v = jnp.zeros((B, D), jnp.float32)
            a = jnp.zeros((B, D), jnp.float32)
            for t in range(nlev):
                v = TAU_MEM * v + proj[:, t, :]
                thresh = bt + asw * a
                s = (v - thresh > 0).astype(jnp.float32)
                v = v * (1.0 - s)
                a = TAU_ADAPT * a + s
                states[nodes[t]] = proj[:, t, :] * s
        nw = nw_ref[...]
        mx = jnp.max(nw, axis=0, keepdims=True)
        e = jnp.exp(nw - mx)
        wts = e / jnp.sum(e, axis=0, keepdims=True)
        mix = jnp.zeros((B, D), jnp.float32)
        for n in range(NUM_NODES):
            mix = mix + wts[n, :][None, :] * states[n]
        mix_ref[...] = mix


def _rmsnorm_kernel(x_ref, mix_ref, w_ref, out_ref):
    xb = x_ref[...]
    m = mix_ref[...]
    y = xb + m
    ms = jnp.mean(y * y, axis=-1, keepdims=True)
    out_ref[...] = y * jax.lax.rsqrt(ms + RMS_EPS) * w_ref[...][None, None, :]


def kernel(x, W_proj, node_weights, rms_weight, base_thresh, adapt_strength):
    B, T, D = x.shape
    padded_T = 1 << int(math.ceil(math.log2(T)))
    assert padded_T == T, "kernel assumes T is a power of two (T=4096)"
    eff_depth = min(TREE_DEPTH, int(math.log2(T)))
    num_leaves = 1 << eff_depth
    sub_seq = T // num_leaves
    half = 8 * sub_seq

    nblk = 8
    mixture = pl.pallas_call(
        _tree_mix_kernel,
        grid=(nblk,),
        in_specs=[
            pl.BlockSpec((B, sub_seq, D), lambda i: (0, i, 0)),
            pl.BlockSpec((D, 2 * D), lambda i: (0, 0)),
            pl.BlockSpec((NUM_NODES, D), lambda i: (0, 0)),
            pl.BlockSpec((TREE_DEPTH, D), lambda i: (0, 0)),
            pl.BlockSpec((TREE_DEPTH, D), lambda i: (0, 0)),
        ],
        out_specs=pl.BlockSpec((B, D), lambda i: (0, 0)),
        out_shape=jax.ShapeDtypeStruct((B, D), jnp.float32),
        scratch_shapes=[pltpu.VMEM((B, 8, D), jnp.float32)],
    )(x[:, :half], W_proj, node_weights, base_thresh, adapt_strength)

    TB = 512
    out = pl.pallas_call(
        _rmsnorm_kernel,
        grid=(B, T // TB),
        in_specs=[
            pl.BlockSpec((1, TB, D), lambda b, j: (b, j, 0)),
            pl.BlockSpec((1, 1, D), lambda b, j: (b, 0, 0)),
            pl.BlockSpec((D,), lambda b, j: (0,)),
        ],
        out_specs=pl.BlockSpec((1, TB, D), lambda b, j: (b, j, 0)),
        out_shape=jax.ShapeDtypeStruct((B, T, D), jnp.float32),
    )(x, mixture.reshape(B, 1, D), rms_weight)
    return out


# speedup vs baseline: 1.7076x; 1.4676x over previous
import jax
import jax.numpy as jnp
from jax.experimental import pallas as pl
from jax.experimental.pallas import tpu as pltpu

TREE_DEPTH = 4
D_MODEL = 1024
NUM_NODES = (1 << TREE_DEPTH) - 1
TAU_MEM = 0.99
TAU_ADAPT = 0.95
RMS_EPS = 1.1920929e-07


def _fused_kernel(x_ref, w_ref, nw_ref, bt_ref, as_ref, rw_ref, out_ref,
                  xfirst_ref, lsum_ref, mix_ref):
    i = pl.program_id(0)
    B = x_ref.shape[0]
    SS = x_ref.shape[1]
    D = x_ref.shape[2]
    sub_seq = 2 * SS

    @pl.when(i < 16)
    def _accum():
        blk = x_ref[...]
        lsum_ref[:, i, :] = jnp.sum(blk, axis=1)
        xfirst_ref[:, pl.ds(i * SS, SS), :] = blk

    @pl.when(i == 16)
    def _tree():
        lsum = lsum_ref[...]
        leaves = lsum.reshape(B, 8, 2, D).sum(axis=2) * (1.0 / sub_seq)
        states = [None] * NUM_NODES
        for n in range(7, 15):
            states[n] = leaves[:, n - 7, :]
        w = w_ref[...]
        for level in range(2, -1, -1):
            lo = (1 << level) - 1
            hi = (1 << (level + 1)) - 1
            nodes = list(range(lo, hi))
            nlev = len(nodes)
            l_st = jnp.stack([states[2 * n + 1] for n in nodes], axis=1)
            r_st = jnp.stack([states[2 * n + 2] for n in nodes], axis=1)
            fused = jnp.concatenate([l_st, r_st], axis=-1)
            fused2 = fused.reshape(B * nlev, 2 * D)
            proj = jax.lax.dot_general(
                fused2, w, (((1,), (1,)), ((), ())),
                preferred_element_type=jnp.float32,
            ).reshape(B, nlev, D)
            bt = bt_ref[level, :]
            asw = as_ref[level, :]
            v = jnp.zeros((B, D), jnp.float32)
            a = jnp.zeros((B, D), jnp.float32)
            for t in range(nlev):
                v = TAU_MEM * v + proj[:, t, :]
                thresh = bt + asw * a
                s = (v - thresh > 0).astype(jnp.float32)
                v = v * (1.0 - s)
                a = TAU_ADAPT * a + s
                states[nodes[t]] = proj[:, t, :] * s
        nw = nw_ref[...]
        mx = jnp.max(nw, axis=0, keepdims=True)
        e = jnp.exp(nw - mx)
        wts = e / jnp.sum(e, axis=0, keepdims=True)
        mix = jnp.zeros((B, D), jnp.float32)
        for n in range(NUM_NODES):
            mix = mix + wts[n, :][None, :] * states[n]
        mix_ref[...] = mix

    def _emit(xb):
        m = mix_ref[...]
        y = xb + m[:, None, :]
        ms = jnp.mean(y * y, axis=-1, keepdims=True)
        out_ref[...] = y * jax.lax.rsqrt(ms + RMS_EPS) * rw_ref[...][None, None, :]

    @pl.when((i >= 16) & (i < 32))
    def _emit_second_half():
        _emit(x_ref[...])

    @pl.when(i >= 32)
    def _emit_first_half():
        j = jnp.maximum(i - 32, 0)
        _emit(xfirst_ref[:, pl.ds(j * SS, SS), :])


def kernel(x, W_proj, node_weights, rms_weight, base_thresh, adapt_strength):
    B, T, D = x.shape
    assert T == 4096, "kernel assumes T=4096"
    SS = T // 32

    def x_idx(i):
        return (0, jnp.minimum(i, 31), 0)

    def out_idx(i):
        j = jnp.where(i < 16, 16, jnp.where(i < 32, i, i - 32))
        return (0, j, 0)

    out = pl.pallas_call(
        _fused_kernel,
        grid=(48,),
        in_specs=[
            pl.BlockSpec((B, SS, D), x_idx),
            pl.BlockSpec((D, 2 * D), lambda i: (0, 0)),
            pl.BlockSpec((NUM_NODES, D), lambda i: (0, 0)),
            pl.BlockSpec((TREE_DEPTH, D), lambda i: (0, 0)),
            pl.BlockSpec((TREE_DEPTH, D), lambda i: (0, 0)),
            pl.BlockSpec((D,), lambda i: (0,)),
        ],
        out_specs=pl.BlockSpec((B, SS, D), out_idx),
        out_shape=jax.ShapeDtypeStruct((B, T, D), jnp.float32),
        scratch_shapes=[
            pltpu.VMEM((B, 16 * SS, D), jnp.float32),
            pltpu.VMEM((B, 16, D), jnp.float32),
            pltpu.VMEM((B, D), jnp.float32),
        ],
    )(x, W_proj, node_weights, base_thresh, adapt_strength, rms_weight)
    return out


# speedup vs baseline: 1.8310x; 1.0722x over previous
import jax
import jax.numpy as jnp
from jax.experimental import pallas as pl
from jax.experimental.pallas import tpu as pltpu

TREE_DEPTH = 4
D_MODEL = 1024
NUM_NODES = (1 << TREE_DEPTH) - 1
TAU_MEM = 0.99
TAU_ADAPT = 0.95
RMS_EPS = 1.1920929e-07


def _fused_kernel(xa_ref, xb_ref, w_ref, nw_ref, bt_ref, as_ref, rw_ref,
                  out_ref, xfirst_ref, lsum_ref, mix_ref):
    i = pl.program_id(0)
    B = xa_ref.shape[0]
    SS = xa_ref.shape[1]
    D = xa_ref.shape[2]
    sub_seq = 2 * SS

    @pl.when(i < 8)
    def _accum():
        blka = xa_ref[...]
        blkb = xb_ref[...]
        lsum_ref[:, i, :] = jnp.sum(blka, axis=1)
        lsum_ref[:, i + 8, :] = jnp.sum(blkb, axis=1)
        xfirst_ref[:, pl.ds(i * SS, SS), :] = blka
        xfirst_ref[:, pl.ds((i + 8) * SS, SS), :] = blkb

    @pl.when(i == 8)
    def _tree():
        lsum = lsum_ref[...]
        leaves = lsum.reshape(B, 8, 2, D).sum(axis=2) * (1.0 / sub_seq)
        states = [None] * NUM_NODES
        for n in range(7, 15):
            states[n] = leaves[:, n - 7, :]
        w = w_ref[...]
        for level in range(2, -1, -1):
            lo = (1 << level) - 1
            hi = (1 << (level + 1)) - 1
            nodes = list(range(lo, hi))
            nlev = len(nodes)
            l_st = jnp.stack([states[2 * n + 1] for n in nodes], axis=1)
            r_st = jnp.stack([states[2 * n + 2] for n in nodes], axis=1)
            fused = jnp.concatenate([l_st, r_st], axis=-1)
            fused2 = fused.reshape(B * nlev, 2 * D)
            proj = jax.lax.dot_general(
                fused2, w, (((1,), (1,)), ((), ())),
                preferred_element_type=jnp.float32,
            ).reshape(B, nlev, D)
            bt = bt_ref[level, :]
            asw = as_ref[level, :]
            v = jnp.zeros((B, D), jnp.float32)
            a = jnp.zeros((B, D), jnp.float32)
            for t in range(nlev):
                v = TAU_MEM * v + proj[:, t, :]
                thresh = bt + asw * a
                s = (v - thresh > 0).astype(jnp.float32)
                v = v * (1.0 - s)
                a = TAU_ADAPT * a + s
                states[nodes[t]] = proj[:, t, :] * s
        nw = nw_ref[...]
        mx = jnp.max(nw, axis=0, keepdims=True)
        e = jnp.exp(nw - mx)
        wts = e / jnp.sum(e, axis=0, keepdims=True)
        mix = jnp.zeros((B, D), jnp.float32)
        for n in range(NUM_NODES):
            mix = mix + wts[n, :][None, :] * states[n]
        mix_ref[...] = mix

    def _emit(xb):
        m = mix_ref[...]
        y = xb + m[:, None, :]
        ms = jnp.mean(y * y, axis=-1, keepdims=True)
        out_ref[...] = y * jax.lax.rsqrt(ms + RMS_EPS) * rw_ref[...][None, None, :]

    @pl.when((i >= 8) & (i < 24))
    def _emit_second_half():
        _emit(xa_ref[...])

    @pl.when(i >= 24)
    def _emit_first_half():
        j = jnp.maximum(i - 24, 0)
        _emit(xfirst_ref[:, pl.ds(j * SS, SS), :])


def kernel(x, W_proj, node_weights, rms_weight, base_thresh, adapt_strength):
    B, T, D = x.shape
    assert T == 4096, "kernel assumes T=4096"
    SS = T // 32

    def xa_idx(i):
        j = jnp.where(i < 8, i, jnp.minimum(i + 8, 31))
        return (0, j, 0)

    def xb_idx(i):
        return (0, jnp.minimum(i + 8, 15), 0)

    def out_idx(i):
        j = jnp.where(i < 8, 16, jnp.where(i < 24, i + 8, i - 24))
        return (0, j, 0)

    out = pl.pallas_call(
        _fused_kernel,
        grid=(40,),
        in_specs=[
            pl.BlockSpec((B, SS, D), xa_idx),
            pl.BlockSpec((B, SS, D), xb_idx),
            pl.BlockSpec((D, 2 * D), lambda i: (0, 0)),
            pl.BlockSpec((NUM_NODES, D), lambda i: (0, 0)),
            pl.BlockSpec((TREE_DEPTH, D), lambda i: (0, 0)),
            pl.BlockSpec((TREE_DEPTH, D), lambda i: (0, 0)),
            pl.BlockSpec((D,), lambda i: (0,)),
        ],
        out_specs=pl.BlockSpec((B, SS, D), out_idx),
        out_shape=jax.ShapeDtypeStruct((B, T, D), jnp.float32),
        scratch_shapes=[
            pltpu.VMEM((B, 16 * SS, D), jnp.float32),
            pltpu.VMEM((B, 16, D), jnp.float32),
            pltpu.VMEM((B, D), jnp.float32),
        ],
    )(x, x, W_proj, node_weights, base_thresh, adapt_strength, rms_weight)
    return out


# speedup vs baseline: 2.0281x; 1.1077x over previous
import jax
import jax.numpy as jnp
from jax.experimental import pallas as pl
from jax.experimental.pallas import tpu as pltpu

TREE_DEPTH = 4
D_MODEL = 1024
NUM_NODES = (1 << TREE_DEPTH) - 1
TAU_MEM = 0.99
TAU_ADAPT = 0.95
RMS_EPS = 1.1920929e-07

SS1 = 256
SS2 = 128
NIN = 8


def _fused_kernel(x_hbm, w_ref, nw_ref, bt_ref, as_ref, rw_ref, out_hbm,
                  xfirst, ring, sem_in1, sem_in2, sem_out1, sem_out2):
    B = 4
    D = D_MODEL
    HALF = 8 * SS1

    def in1_copy(j):
        return pltpu.make_async_copy(
            x_hbm.at[:, pl.ds(j * SS1, SS1), :],
            xfirst.at[:, pl.ds(j * SS1, SS1), :],
            sem_in1.at[j])

    def in2_copy(k):
        return pltpu.make_async_copy(
            x_hbm.at[:, pl.ds(HALF + k * SS2, SS2), :],
            ring.at[k % NIN],
            sem_in2.at[k % NIN])

    def out1_copy(j):
        return pltpu.make_async_copy(
            xfirst.at[:, pl.ds(j * SS1, SS1), :],
            out_hbm.at[:, pl.ds(j * SS1, SS1), :],
            sem_out1.at[j])

    def out2_copy(k):
        return pltpu.make_async_copy(
            ring.at[k % NIN],
            out_hbm.at[:, pl.ds(HALF + k * SS2, SS2), :],
            sem_out2.at[k % NIN])

    for j in range(8):
        in1_copy(j).start()
    for k in range(NIN):
        in2_copy(k).start()
    for j in range(8):
        in1_copy(j).wait()

    leaves = jnp.sum(
        xfirst[...].reshape(B, 8, SS1, D), axis=2) * (1.0 / SS1)
    states = [None] * NUM_NODES
    for n in range(7, 15):
        states[n] = leaves[:, n - 7, :]
    w = w_ref[...]
    for level in range(2, -1, -1):
        lo = (1 << level) - 1
        hi = (1 << (level + 1)) - 1
        nodes = list(range(lo, hi))
        nlev = len(nodes)
        l_st = jnp.stack([states[2 * n + 1] for n in nodes], axis=1)
        r_st = jnp.stack([states[2 * n + 2] for n in nodes], axis=1)
        fused = jnp.concatenate([l_st, r_st], axis=-1)
        fused2 = fused.reshape(B * nlev, 2 * D)
        proj = jax.lax.dot_general(
            fused2, w, (((1,), (1,)), ((), ())),
            preferred_element_type=jnp.float32,
        ).reshape(B, nlev, D)
        bt = bt_ref[level, :]
        asw = as_ref[level, :]
        v = jnp.zeros((B, D), jnp.float32)
        a = jnp.zeros((B, D), jnp.float32)
        for t in range(nlev):
            v = TAU_MEM * v + proj[:, t, :]
            thresh = bt + asw * a
            s = (v - thresh > 0).astype(jnp.float32)
            v = v * (1.0 - s)
            a = TAU_ADAPT * a + s
            states[nodes[t]] = proj[:, t, :] * s
    nw = nw_ref[...]
    mx = jnp.max(nw, axis=0, keepdims=True)
    e = jnp.exp(nw - mx)
    wts = e / jnp.sum(e, axis=0, keepdims=True)
    mix = jnp.zeros((B, D), jnp.float32)
    for n in range(NUM_NODES):
        mix = mix + wts[n, :][None, :] * states[n]
    rw = rw_ref[...]

    def rms(y):
        ms = jnp.mean(y * y, axis=-1, keepdims=True)
        return y * jax.lax.rsqrt(ms + RMS_EPS) * rw[None, None, :]

    for j in range(8):
        xblk = xfirst[:, pl.ds(j * SS1, SS1), :]
        xfirst[:, pl.ds(j * SS1, SS1), :] = rms(xblk + mix[:, None, :])
        out1_copy(j).start()

    for k in range(16):
        in2_copy(k).wait()
        blk = ring[k % NIN]
        ring[k % NIN] = rms(blk + mix[:, None, :])
        out2_copy(k).start()
        if k >= 1 and (k - 1) + NIN < 16:
            out2_copy(k - 1).wait()
            in2_copy(k - 1 + NIN).start()
    for k in range(NIN, 16):
        out2_copy(k).wait()
    for j in range(8):
        out1_copy(j).wait()


def kernel(x, W_proj, node_weights, rms_weight, base_thresh, adapt_strength):
    B, T, D = x.shape
    assert T == 4096 and D == D_MODEL, "kernel assumes T=4096, D=1024"

    out = pl.pallas_call(
        _fused_kernel,
        in_specs=[
            pl.BlockSpec(memory_space=pl.ANY),
            pl.BlockSpec(memory_space=pltpu.VMEM),
            pl.BlockSpec(memory_space=pltpu.VMEM),
            pl.BlockSpec(memory_space=pltpu.VMEM),
            pl.BlockSpec(memory_space=pltpu.VMEM),
            pl.BlockSpec(memory_space=pltpu.VMEM),
        ],
        out_specs=pl.BlockSpec(memory_space=pl.ANY),
        out_shape=jax.ShapeDtypeStruct((B, T, D), jnp.float32),
        scratch_shapes=[
            pltpu.VMEM((B, 8 * SS1, D), jnp.float32),
            pltpu.VMEM((NIN, B, SS2, D), jnp.float32),
            pltpu.SemaphoreType.DMA((8,)),
            pltpu.SemaphoreType.DMA((NIN,)),
            pltpu.SemaphoreType.DMA((8,)),
            pltpu.SemaphoreType.DMA((NIN,)),
        ],
        compiler_params=pltpu.CompilerParams(vmem_limit_bytes=62 * 1024 * 1024),
    )(x, W_proj, node_weights, base_thresh, adapt_strength, rms_weight)
    return out
